# Initial kernel scaffold; baseline (speedup 1.0000x reference)
import jax
import jax.numpy as jnp
from jax.experimental import pallas as pl


def kernel(words, context, vocab, dense_w, dense_b):
    raise NotImplementedError("write your pallas kernel here")


# speedup vs baseline: 5.2615x; 5.2615x over previous
import functools

import jax
import jax.numpy as jnp
from jax import lax
from jax.experimental import pallas as pl
from jax.experimental.pallas import tpu as pltpu
from jax.experimental.pallas import tpu_sc as plsc

E = 16
LW = 50
LC = 100
NC = 2
NS = 16
NWORKERS = NC * NS
CB = 16


def _sc_body(nchunk, words_ref, ctx_ref, vocab_ref, w_ref, b_ref, out_ref,
             widx0, widx1, cidx0, cidx1, wrows0, wrows1, crows0, crows1,
             outv, pbuf, wbv, bbv, si0, si1, sr0, sr1):
    widx = (widx0, widx1)
    cidx = (cidx0, cidx1)
    wrows = (wrows0, wrows1)
    crows = (crows0, crows1)
    si = (si0, si1)
    sr = (sr0, sr1)

    wid = lax.axis_index("s") * NC + lax.axis_index("c")
    rows_per_worker = nchunk * CB
    wbase0 = wid * (rows_per_worker * LW)
    cbase0 = wid * (rows_per_worker * LC)

    pltpu.sync_copy(w_ref, wbv)
    pltpu.sync_copy(b_ref, bbv)
    wv = wbv[...]
    bv = bbv[...]

    def idx_start(g, b):
        pltpu.make_async_copy(
            words_ref.at[pl.ds(wbase0 + g * (CB * LW), CB * LW)],
            widx[b], si[b]).start()
        pltpu.make_async_copy(
            ctx_ref.at[pl.ds(cbase0 + g * (CB * LC), CB * LC)],
            cidx[b], si[b]).start()

    def idx_wait(b):
        pltpu.make_async_copy(
            words_ref.at[pl.ds(0, CB * LW)], widx[b], si[b]).wait()
        pltpu.make_async_copy(
            ctx_ref.at[pl.ds(0, CB * LC)], cidx[b], si[b]).wait()

    def rows_start(b):
        pltpu.make_async_copy(vocab_ref.at[widx[b]], wrows[b], sr[b]).start()
        pltpu.make_async_copy(vocab_ref.at[cidx[b]], crows[b], sr[b]).start()

    def rows_wait(b):
        pltpu.make_async_copy(vocab_ref.at[widx[b]], wrows[b], sr[b]).wait()
        pltpu.make_async_copy(vocab_ref.at[cidx[b]], crows[b], sr[b]).wait()

    def compute(g, b):
        wr = wrows[b]
        cr = crows[b]
        z16 = jnp.zeros((E,), jnp.float32)

        def row_body(r, carry):
            woff = r * LW
            coff = r * LC

            def wstep(j, accs):
                i = woff + j * 5
                return tuple(a + wr[i + t] for t, a in enumerate(accs))

            aw = lax.fori_loop(0, LW // 5, wstep, (z16,) * 5)
            wsum = ((aw[0] + aw[1]) + (aw[2] + aw[3])) + aw[4]

            def cstep(j, accs):
                i = coff + j * 5
                return tuple(a + cr[i + t] for t, a in enumerate(accs))

            ac = lax.fori_loop(0, LC // 5, cstep, (z16,) * 5)
            csum = ((ac[0] + ac[1]) + (ac[2] + ac[3])) + ac[4]

            pbuf[pl.ds(r * E, E)] = wsum * csum
            return carry

        lax.fori_loop(0, CB, row_body, 0)

        lane = lax.iota(jnp.int32, E)
        base = lane * E
        acc = z16
        for c in range(E):
            acc = acc + plsc.load_gather(pbuf, [base + c])
        zv = acc * wv + bv
        ov = 1.0 / (1.0 + jnp.exp(-zv))
        outv[pl.ds(g * CB, CB)] = ov

    idx_start(0, 0)
    idx_wait(0)
    rows_start(0)
    idx_start(1, 1)

    def outer(gg, carry):
        for b in (0, 1):
            g = gg * 2 + b
            nb = 1 - b
            rows_wait(b)

            @pl.when(g + 2 < nchunk)
            def _():
                idx_start(g + 2, b)

            @pl.when(g + 1 < nchunk)
            def _():
                idx_wait(nb)
                rows_start(nb)

            compute(g, b)
        return carry

    lax.fori_loop(0, nchunk // 2, outer, 0)

    pltpu.sync_copy(
        outv, out_ref.at[pl.ds(wid * rows_per_worker, rows_per_worker)])


@functools.partial(jax.jit, static_argnames=())
def kernel(words, context, vocab, dense_w, dense_b):
    B, lw = words.shape
    lc = context.shape[1] * context.shape[2]
    assert lw == LW and lc == LC and vocab.shape[1] == E
    assert B % (NWORKERS * CB) == 0
    nchunk = B // (NWORKERS * CB)

    words_f = jnp.asarray(words, jnp.int32).reshape(-1)
    ctx_f = jnp.asarray(context, jnp.int32).reshape(-1)
    vocab = jnp.asarray(vocab, jnp.float32)
    w16 = jnp.broadcast_to(
        jnp.asarray(dense_w, jnp.float32).reshape(-1)[:1], (E,))
    b16 = jnp.broadcast_to(
        jnp.asarray(dense_b, jnp.float32).reshape(-1)[:1], (E,))

    mesh = plsc.VectorSubcoreMesh(
        core_axis_name="c", subcore_axis_name="s",
        num_cores=NC, num_subcores=NS)
    run = pl.kernel(
        functools.partial(_sc_body, nchunk),
        out_type=jax.ShapeDtypeStruct((B,), jnp.float32),
        mesh=mesh,
        compiler_params=pltpu.CompilerParams(
            needs_layout_passes=False, use_tc_tiling_on_sc=False),
        scratch_types=[
            pltpu.VMEM((CB * LW,), jnp.int32),
            pltpu.VMEM((CB * LW,), jnp.int32),
            pltpu.VMEM((CB * LC,), jnp.int32),
            pltpu.VMEM((CB * LC,), jnp.int32),
            pltpu.VMEM((CB * LW, E), jnp.float32),
            pltpu.VMEM((CB * LW, E), jnp.float32),
            pltpu.VMEM((CB * LC, E), jnp.float32),
            pltpu.VMEM((CB * LC, E), jnp.float32),
            pltpu.VMEM((B // NWORKERS,), jnp.float32),
            pltpu.VMEM((CB * E,), jnp.float32),
            pltpu.VMEM((E,), jnp.float32),
            pltpu.VMEM((E,), jnp.float32),
            pltpu.SemaphoreType.DMA,
            pltpu.SemaphoreType.DMA,
            pltpu.SemaphoreType.DMA,
            pltpu.SemaphoreType.DMA,
        ],
    )
    out = run(words_f, ctx_f, vocab, w16, b16)
    return out.reshape(B, 1)


# speedup vs baseline: 6.0516x; 1.1502x over previous
import functools

import jax
import jax.numpy as jnp
from jax import lax
from jax.experimental import pallas as pl
from jax.experimental.pallas import tpu as pltpu
from jax.experimental.pallas import tpu_sc as plsc

E = 16
LW = 50
LC = 100
NC = 2
NS = 16
NWORKERS = NC * NS


def _sc_body(rpw, words_ref, ctx_ref, vocab_ref, w_ref, b_ref, out_ref,
             widx, cidx, wacc, cacc, outv, pbuf, wbv, bbv, s_idx, s_acc):
    wid = lax.axis_index("s") * NC + lax.axis_index("c")
    base0 = wid * rpw

    pltpu.sync_copy(w_ref, wbv)
    pltpu.sync_copy(b_ref, bbv)

    pltpu.make_async_copy(
        words_ref.at[:, pl.ds(base0, rpw)], widx, s_idx).start()
    pltpu.make_async_copy(
        ctx_ref.at[:, pl.ds(base0, rpw)], cidx, s_idx).start()

    z16 = jnp.zeros((E,), jnp.float32)

    def zero_body(i, carry):
        wacc[i] = z16
        cacc[i] = z16
        return carry

    lax.fori_loop(0, rpw, zero_body, 0)

    pltpu.make_async_copy(
        words_ref.at[:, pl.ds(0, rpw)], widx, s_idx).wait()
    pltpu.make_async_copy(
        ctx_ref.at[:, pl.ds(0, rpw)], cidx, s_idx).wait()

    def wg_body(j, carry):
        pltpu.async_copy(vocab_ref.at[widx.at[j]], wacc, s_acc, add=True)
        return carry

    lax.fori_loop(0, LW, wg_body, 0)

    def cg_body(j, carry):
        pltpu.async_copy(vocab_ref.at[cidx.at[j]], cacc, s_acc, add=True)
        return carry

    lax.fori_loop(0, LC, cg_body, 0)

    def drain_w(j, carry):
        pltpu.make_async_copy(vocab_ref.at[widx.at[0]], wacc, s_acc).wait()
        return carry

    lax.fori_loop(0, LW, drain_w, 0)

    def drain_c(j, carry):
        pltpu.make_async_copy(vocab_ref.at[cidx.at[0]], cacc, s_acc).wait()
        return carry

    lax.fori_loop(0, LC, drain_c, 0)

    wv = wbv[...]
    bv = bbv[...]
    lane = lax.iota(jnp.int32, E)
    gbase = lane * E

    def group_body(g, carry):
        def prod_body(i, carry2):
            r = g * E + i
            pbuf[pl.ds(i * E, E)] = wacc[r] * cacc[r]
            return carry2

        lax.fori_loop(0, E, prod_body, 0)

        acc = z16
        for c in range(E):
            acc = acc + plsc.load_gather(pbuf, [gbase + c])
        zv = acc * wv + bv
        ov = 1.0 / (1.0 + jnp.exp(-zv))
        outv[pl.ds(g * E, E)] = ov
        return carry

    lax.fori_loop(0, rpw // E, group_body, 0)

    pltpu.sync_copy(outv, out_ref.at[pl.ds(base0, rpw)])


@functools.partial(jax.jit, static_argnames=())
def kernel(words, context, vocab, dense_w, dense_b):
    B, lw = words.shape
    lc = context.shape[1] * context.shape[2]
    assert lw == LW and lc == LC and vocab.shape[1] == E
    assert B % (NWORKERS * E) == 0
    rpw = B // NWORKERS

    words_t = jnp.asarray(words, jnp.int32).T
    ctx_t = jnp.asarray(context, jnp.int32).transpose(2, 1, 0).reshape(LC, B)
    vocab = jnp.asarray(vocab, jnp.float32)
    w16 = jnp.broadcast_to(
        jnp.asarray(dense_w, jnp.float32).reshape(-1)[:1], (E,))
    b16 = jnp.broadcast_to(
        jnp.asarray(dense_b, jnp.float32).reshape(-1)[:1], (E,))

    mesh = plsc.VectorSubcoreMesh(
        core_axis_name="c", subcore_axis_name="s",
        num_cores=NC, num_subcores=NS)
    run = pl.kernel(
        functools.partial(_sc_body, rpw),
        out_type=jax.ShapeDtypeStruct((B,), jnp.float32),
        mesh=mesh,
        compiler_params=pltpu.CompilerParams(
            needs_layout_passes=False, use_tc_tiling_on_sc=False),
        scratch_types=[
            pltpu.VMEM((LW, rpw), jnp.int32),
            pltpu.VMEM((LC, rpw), jnp.int32),
            pltpu.VMEM((rpw, E), jnp.float32),
            pltpu.VMEM((rpw, E), jnp.float32),
            pltpu.VMEM((rpw,), jnp.float32),
            pltpu.VMEM((E * E,), jnp.float32),
            pltpu.VMEM((E,), jnp.float32),
            pltpu.VMEM((E,), jnp.float32),
            pltpu.SemaphoreType.DMA,
            pltpu.SemaphoreType.DMA,
        ],
    )
    out = run(words_t, ctx_t, vocab, w16, b16)
    return out.reshape(B, 1)
